# baseline (speedup 1.0000x reference)
import jax
import jax.numpy as jnp
from jax import lax
from jax.experimental import pallas as pl
from jax.experimental.pallas import tpu as pltpu
from jax.experimental.pallas import tpu_sc as plsc

B, C, H, W = 8, 96, 224, 224
K = 3
CB = 16
NC_ = C // CB
ROWS_PER_PLANE = 16
PLANE_MINOR = (H * W) // ROWS_PER_PLANE
NPLANES = B * 6


def _mean_body(f1_ref, f2_ref, m1_ref, m2_ref):
    m1_ref[...] = jnp.mean(f1_ref[...], axis=(2, 3)).reshape(1, 1, 1, CB)
    m2_ref[...] = jnp.mean(f2_ref[...], axis=(2, 3)).reshape(1, 1, 1, CB)


def _channel_means(F1, F2):
    out_sd = jax.ShapeDtypeStruct((B, NC_, 1, CB), jnp.float32)
    m1, m2 = pl.pallas_call(
        _mean_body,
        grid=(B, NC_),
        in_specs=[
            pl.BlockSpec((1, CB, H, W), lambda b, c: (b, c, 0, 0)),
            pl.BlockSpec((1, CB, H, W), lambda b, c: (b, c, 0, 0)),
        ],
        out_specs=[
            pl.BlockSpec((1, 1, 1, CB), lambda b, c: (b, c, 0, 0)),
            pl.BlockSpec((1, 1, 1, CB), lambda b, c: (b, c, 0, 0)),
        ],
        out_shape=[out_sd, out_sd],
        compiler_params=pltpu.CompilerParams(
            dimension_semantics=(pltpu.PARALLEL, pltpu.PARALLEL)),
    )(F1, F2)
    return m1.reshape(B, C), m2.reshape(B, C)


def _gating_body(fr_ref, x1_ref, x2_ref, wfr_ref,
                 rows_ref, w1_ref, w2_ref):
    fr_r = fr_ref[...].astype(jnp.bfloat16).astype(jnp.float32)
    wfr_r = wfr_ref[...].astype(jnp.bfloat16).astype(jnp.float32)
    mfr = jnp.mean(fr_r, axis=(2, 3))
    frp = (mfr[:, :, None] * wfr_r[None, :, :]).sum(axis=1)
    iota = lax.broadcasted_iota(jnp.int32, (B, C), 1)

    def top3(dist, w_ref):
        d = dist
        vals, idxs = [], []
        for _ in range(K):
            v = jnp.max(d, axis=1, keepdims=True)
            hit = d == v
            idx = jnp.min(jnp.where(hit, iota, C), axis=1, keepdims=True)
            vals.append(v)
            idxs.append(idx)
            d = jnp.where(iota == idx, -jnp.inf, d)
        tv = jnp.concatenate(vals, axis=1)
        ti = jnp.concatenate(idxs, axis=1)
        e = jnp.exp(tv - tv[:, :1])
        w_ref[...] = e / jnp.sum(e, axis=1, keepdims=True)
        return ti

    ti1 = top3(-jnp.abs(frp - x1_ref[...]), w1_ref)
    ti2 = top3(-jnp.abs(frp - x2_ref[...]), w2_ref)
    boff = lax.broadcasted_iota(jnp.int32, (B, K), 0) * C
    pad = jnp.zeros((B, 2), jnp.int32)
    rows_ref[...] = jnp.concatenate([boff + ti1, boff + ti2, pad], axis=1)


def _gating(fr, x1, x2, wfr_t):
    return pl.pallas_call(
        _gating_body,
        out_shape=[
            jax.ShapeDtypeStruct((B, 8), jnp.int32),
            jax.ShapeDtypeStruct((B, K), jnp.float32),
            jax.ShapeDtypeStruct((B, K), jnp.float32),
        ],
    )(fr, x1, x2, wfr_t)


def _sc_gather_body(rows_hbm, f1_hbm, f2_hbm, out_hbm, idx_v, plane_v, sem):
    wid = lax.axis_index("s") * 2 + lax.axis_index("c")
    pltpu.sync_copy(rows_hbm, idx_v)
    iota16 = lax.iota(jnp.int32, 16)
    c0 = idx_v[pl.ds(0, 16)]
    c1 = idx_v[pl.ds(16, 16)]
    c2 = idx_v[pl.ds(32, 16)]
    c3 = idx_v[pl.ds(48, 16)]

    def do_plane(p):
        b = p // 6
        slot = p - b * 6
        q = b * 8 + slot
        chunk = q // 16
        lane = q - chunk * 16
        chv = jnp.full((16,), chunk, jnp.int32)
        cv = jnp.where(chv == 0, c0,
                       jnp.where(chv == 1, c1,
                                 jnp.where(chv == 2, c2, c3)))
        lv = jnp.full((16,), lane, jnp.int32)
        row = jnp.sum(jnp.where(iota16 == lv, cv, 0))

        @pl.when(slot < K)
        def _():
            pltpu.async_copy(f1_hbm.at[row], plane_v, sem).wait()

        @pl.when(slot >= K)
        def _():
            pltpu.async_copy(f2_hbm.at[row], plane_v, sem).wait()

        pltpu.sync_copy(plane_v, out_hbm.at[p])

    do_plane(wid)

    @pl.when(wid < NPLANES - 32)
    def _():
        do_plane(wid + 32)


def _sc_gather(rows64, F1r, F2r):
    f = pl.kernel(
        _sc_gather_body,
        out_type=jax.ShapeDtypeStruct((NPLANES, H, W), jnp.float32),
        mesh=plsc.VectorSubcoreMesh(core_axis_name="c", subcore_axis_name="s"),
        compiler_params=pltpu.CompilerParams(needs_layout_passes=False),
        scratch_types=[
            pltpu.VMEM((64,), jnp.int32),
            pltpu.VMEM((H, W), jnp.float32),
            pltpu.SemaphoreType.DMA,
        ],
    )
    return f(rows64, F1r, F2r)


def _fuse_body(cat_ref, w1_ref, w2_ref, wl_ref, bl_ref, wa1_ref, wa2_ref,
               out_ref, catp_ref, fusedp_ref, h_ref):
    b = pl.program_id(0)
    catp_ref[...] = jnp.zeros_like(catp_ref)
    fusedp_ref[...] = jnp.zeros_like(fusedp_ref)

    for k in range(6):
        catp_ref[k, 1:H + 1, 1:W + 1] = cat_ref[0, k]

    acc1 = [jnp.full((H, W), bl_ref[o], dtype=jnp.float32) for o in range(3)]
    for i in range(6):
        ws = w1_ref[b, i] if i < K else w2_ref[b, i - K]
        for dy in range(3):
            for dx in range(3):
                s = catp_ref[i, dy:dy + H, dx:dx + W]
                for o in range(3):
                    acc1[o] = acc1[o] + s * (wl_ref[(o * 6 + i) * 9 + dy * 3 + dx] * ws)
    for o in range(3):
        fusedp_ref[o, 1:H + 1, 1:W + 1] = acc1[o]

    acc2 = [jnp.zeros((H, W), dtype=jnp.float32) for _ in range(8)]
    for i in range(3):
        for dy in range(3):
            for dx in range(3):
                s = fusedp_ref[i, dy:dy + H, dx:dx + W]
                for o in range(8):
                    acc2[o] = acc2[o] + s * wa1_ref[(o * 3 + i) * 9 + dy * 3 + dx]
    for o in range(8):
        h_ref[o] = jnp.maximum(acc2[o], 0.0)

    acc3 = [jnp.zeros((H, W), dtype=jnp.float32) for _ in range(3)]
    for i in range(8):
        hv = h_ref[i]
        for o in range(3):
            acc3[o] = acc3[o] + hv * wa2_ref[o * 8 + i]
    for o in range(3):
        out_ref[0, o] = jax.nn.sigmoid(acc3[o])


def _fuse(catg, w1, w2, wl, bl, wa1, wa2):
    return pl.pallas_call(
        _fuse_body,
        grid=(B,),
        in_specs=[
            pl.BlockSpec((1, 6, H, W), lambda b: (b, 0, 0, 0)),
            pl.BlockSpec(memory_space=pltpu.SMEM),
            pl.BlockSpec(memory_space=pltpu.SMEM),
            pl.BlockSpec(memory_space=pltpu.SMEM),
            pl.BlockSpec(memory_space=pltpu.SMEM),
            pl.BlockSpec(memory_space=pltpu.SMEM),
            pl.BlockSpec(memory_space=pltpu.SMEM),
        ],
        out_specs=pl.BlockSpec((1, 3, H, W), lambda b: (b, 0, 0, 0)),
        out_shape=jax.ShapeDtypeStruct((B, 3, H, W), jnp.float32),
        scratch_shapes=[
            pltpu.VMEM((6, H + 2, W + 2), jnp.float32),
            pltpu.VMEM((3, H + 2, W + 2), jnp.float32),
            pltpu.VMEM((8, H, W), jnp.float32),
        ],
    )(catg, w1, w2, wl, bl, wa1, wa2)


def kernel(F1, F2, fr, W_fr, W_last, b_last, W_att1, W_att2):
    x1, x2 = _channel_means(F1, F2)
    wfr_t = W_fr.reshape(C, 3).T
    rows, w1, w2 = _gating(fr, x1, x2, wfr_t)
    F1r = F1.reshape(B * C, H, W)
    F2r = F2.reshape(B * C, H, W)
    catg = _sc_gather(rows.reshape(-1), F1r, F2r)
    catg = catg.reshape(B, 6, H, W)
    wl = W_last.reshape(-1)
    wa1 = W_att1.reshape(-1)
    wa2 = W_att2.reshape(-1)
    return _fuse(catg, w1, w2, wl, b_last, wa1, wa2)


# speedup vs baseline: 1.1380x; 1.1380x over previous
import jax
import jax.numpy as jnp
from jax import lax
from jax.experimental import pallas as pl
from jax.experimental.pallas import tpu as pltpu
from jax.experimental.pallas import tpu_sc as plsc

B, C, H, W = 8, 96, 224, 224
K = 3
CB = 16
NC_ = C // CB
ROWS_PER_PLANE = 16
PLANE_MINOR = (H * W) // ROWS_PER_PLANE
NPLANES = B * 6


def _mean_body(f1_ref, f2_ref, m1_ref, m2_ref):
    m1_ref[...] = jnp.mean(f1_ref[...], axis=(2, 3)).reshape(1, 1, 1, CB)
    m2_ref[...] = jnp.mean(f2_ref[...], axis=(2, 3)).reshape(1, 1, 1, CB)


def _channel_means(F1, F2):
    out_sd = jax.ShapeDtypeStruct((B, NC_, 1, CB), jnp.float32)
    m1, m2 = pl.pallas_call(
        _mean_body,
        grid=(B, NC_),
        in_specs=[
            pl.BlockSpec((1, CB, H, W), lambda b, c: (b, c, 0, 0)),
            pl.BlockSpec((1, CB, H, W), lambda b, c: (b, c, 0, 0)),
        ],
        out_specs=[
            pl.BlockSpec((1, 1, 1, CB), lambda b, c: (b, c, 0, 0)),
            pl.BlockSpec((1, 1, 1, CB), lambda b, c: (b, c, 0, 0)),
        ],
        out_shape=[out_sd, out_sd],
        compiler_params=pltpu.CompilerParams(
            dimension_semantics=(pltpu.PARALLEL, pltpu.PARALLEL)),
    )(F1, F2)
    return m1.reshape(B, C), m2.reshape(B, C)


def _gating_body(fr_ref, x1_ref, x2_ref, wfr_ref,
                 rows_ref, w1_ref, w2_ref):
    fr_r = fr_ref[...].astype(jnp.bfloat16).astype(jnp.float32)
    wfr_r = wfr_ref[...].astype(jnp.bfloat16).astype(jnp.float32)
    mfr = jnp.mean(fr_r, axis=(2, 3))
    frp = (mfr[:, :, None] * wfr_r[None, :, :]).sum(axis=1)
    iota = lax.broadcasted_iota(jnp.int32, (B, C), 1)

    def top3(dist, w_ref):
        d = dist
        vals, idxs = [], []
        for _ in range(K):
            v = jnp.max(d, axis=1, keepdims=True)
            hit = d == v
            idx = jnp.min(jnp.where(hit, iota, C), axis=1, keepdims=True)
            vals.append(v)
            idxs.append(idx)
            d = jnp.where(iota == idx, -jnp.inf, d)
        tv = jnp.concatenate(vals, axis=1)
        ti = jnp.concatenate(idxs, axis=1)
        e = jnp.exp(tv - tv[:, :1])
        w_ref[...] = e / jnp.sum(e, axis=1, keepdims=True)
        return ti

    ti1 = top3(-jnp.abs(frp - x1_ref[...]), w1_ref)
    ti2 = top3(-jnp.abs(frp - x2_ref[...]), w2_ref)
    boff = lax.broadcasted_iota(jnp.int32, (B, K), 0) * C
    pad = jnp.zeros((B, 2), jnp.int32)
    rows_ref[...] = jnp.concatenate([boff + ti1, boff + ti2, pad], axis=1)


def _gating(fr, x1, x2, wfr_t):
    return pl.pallas_call(
        _gating_body,
        out_shape=[
            jax.ShapeDtypeStruct((B, 8), jnp.int32),
            jax.ShapeDtypeStruct((B, K), jnp.float32),
            jax.ShapeDtypeStruct((B, K), jnp.float32),
        ],
    )(fr, x1, x2, wfr_t)


def _sc_gather_body(rows_hbm, f1_hbm, f2_hbm, out_hbm, idx_v, plane_v, sem):
    wid = lax.axis_index("s") * 2 + lax.axis_index("c")
    pltpu.sync_copy(rows_hbm, idx_v)
    iota16 = lax.iota(jnp.int32, 16)
    c0 = idx_v[pl.ds(0, 16)]
    c1 = idx_v[pl.ds(16, 16)]
    c2 = idx_v[pl.ds(32, 16)]
    c3 = idx_v[pl.ds(48, 16)]

    def do_plane(p):
        b = p // 6
        slot = p - b * 6
        q = b * 8 + slot
        chunk = q // 16
        lane = q - chunk * 16
        chv = jnp.full((16,), chunk, jnp.int32)
        cv = jnp.where(chv == 0, c0,
                       jnp.where(chv == 1, c1,
                                 jnp.where(chv == 2, c2, c3)))
        lv = jnp.full((16,), lane, jnp.int32)
        row = jnp.sum(jnp.where(iota16 == lv, cv, 0))

        @pl.when(slot < K)
        def _():
            pltpu.async_copy(f1_hbm.at[row], plane_v, sem).wait()

        @pl.when(slot >= K)
        def _():
            pltpu.async_copy(f2_hbm.at[row], plane_v, sem).wait()

        pltpu.sync_copy(plane_v, out_hbm.at[p])

    do_plane(wid)

    @pl.when(wid < NPLANES - 32)
    def _():
        do_plane(wid + 32)


def _sc_gather(rows64, F1r, F2r):
    f = pl.kernel(
        _sc_gather_body,
        out_type=jax.ShapeDtypeStruct((NPLANES, H, W), jnp.float32),
        mesh=plsc.VectorSubcoreMesh(core_axis_name="c", subcore_axis_name="s"),
        compiler_params=pltpu.CompilerParams(needs_layout_passes=False),
        scratch_types=[
            pltpu.VMEM((64,), jnp.int32),
            pltpu.VMEM((H, W), jnp.float32),
            pltpu.SemaphoreType.DMA,
        ],
    )
    return f(rows64, F1r, F2r)


def _shift3(x):
    zcol = jnp.zeros((H, 1), jnp.float32)
    return (jnp.concatenate([zcol, x[:, :W - 1]], axis=1),
            x,
            jnp.concatenate([x[:, 1:], zcol], axis=1))


def _fuse_body(cat_ref, w1_ref, w2_ref, wl_ref, bl_ref, wa1_ref, wa2_ref,
               out_ref, catp_ref, fusedp_ref, h_ref):
    b = pl.program_id(0)
    catp_ref[:, :, 0:1, :] = jnp.zeros((6, 3, 1, W), jnp.float32)
    catp_ref[:, :, H + 1:H + 2, :] = jnp.zeros((6, 3, 1, W), jnp.float32)
    fusedp_ref[:, :, 0:1, :] = jnp.zeros((3, 3, 1, W), jnp.float32)
    fusedp_ref[:, :, H + 1:H + 2, :] = jnp.zeros((3, 3, 1, W), jnp.float32)

    for k in range(6):
        s0, s1, s2 = _shift3(cat_ref[0, k])
        catp_ref[k, 0, 1:H + 1, :] = s0
        catp_ref[k, 1, 1:H + 1, :] = s1
        catp_ref[k, 2, 1:H + 1, :] = s2

    acc1 = [jnp.full((H, W), bl_ref[o], dtype=jnp.float32) for o in range(3)]
    for i in range(6):
        ws = w1_ref[b, i] if i < K else w2_ref[b, i - K]
        for dy in range(3):
            for dx in range(3):
                s = catp_ref[i, dx, dy:dy + H, :]
                for o in range(3):
                    acc1[o] = acc1[o] + s * (wl_ref[(o * 6 + i) * 9 + dy * 3 + dx] * ws)
    for o in range(3):
        s0, s1, s2 = _shift3(acc1[o])
        fusedp_ref[o, 0, 1:H + 1, :] = s0
        fusedp_ref[o, 1, 1:H + 1, :] = s1
        fusedp_ref[o, 2, 1:H + 1, :] = s2

    acc2 = [jnp.zeros((H, W), dtype=jnp.float32) for _ in range(8)]
    for i in range(3):
        for dy in range(3):
            for dx in range(3):
                s = fusedp_ref[i, dx, dy:dy + H, :]
                for o in range(8):
                    acc2[o] = acc2[o] + s * wa1_ref[(o * 3 + i) * 9 + dy * 3 + dx]
    for o in range(8):
        h_ref[o] = jnp.maximum(acc2[o], 0.0)

    acc3 = [jnp.zeros((H, W), dtype=jnp.float32) for _ in range(3)]
    for i in range(8):
        hv = h_ref[i]
        for o in range(3):
            acc3[o] = acc3[o] + hv * wa2_ref[o * 8 + i]
    for o in range(3):
        out_ref[0, o] = jax.nn.sigmoid(acc3[o])


def _fuse(catg, w1, w2, wl, bl, wa1, wa2):
    return pl.pallas_call(
        _fuse_body,
        grid=(B,),
        in_specs=[
            pl.BlockSpec((1, 6, H, W), lambda b: (b, 0, 0, 0)),
            pl.BlockSpec(memory_space=pltpu.SMEM),
            pl.BlockSpec(memory_space=pltpu.SMEM),
            pl.BlockSpec(memory_space=pltpu.SMEM),
            pl.BlockSpec(memory_space=pltpu.SMEM),
            pl.BlockSpec(memory_space=pltpu.SMEM),
            pl.BlockSpec(memory_space=pltpu.SMEM),
        ],
        out_specs=pl.BlockSpec((1, 3, H, W), lambda b: (b, 0, 0, 0)),
        out_shape=jax.ShapeDtypeStruct((B, 3, H, W), jnp.float32),
        scratch_shapes=[
            pltpu.VMEM((6, 3, H + 2, W), jnp.float32),
            pltpu.VMEM((3, 3, H + 2, W), jnp.float32),
            pltpu.VMEM((8, H, W), jnp.float32),
        ],
    )(catg, w1, w2, wl, bl, wa1, wa2)


def kernel(F1, F2, fr, W_fr, W_last, b_last, W_att1, W_att2):
    x1, x2 = _channel_means(F1, F2)
    wfr_t = W_fr.reshape(C, 3).T
    rows, w1, w2 = _gating(fr, x1, x2, wfr_t)
    F1r = F1.reshape(B * C, H, W)
    F2r = F2.reshape(B * C, H, W)
    catg = _sc_gather(rows.reshape(-1), F1r, F2r)
    catg = catg.reshape(B, 6, H, W)
    wl = W_last.reshape(-1)
    wa1 = W_att1.reshape(-1)
    wa2 = W_att2.reshape(-1)
    return _fuse(catg, w1, w2, wl, b_last, wa1, wa2)
